# Initial kernel scaffold; baseline (speedup 1.0000x reference)
import jax
import jax.numpy as jnp
from jax.experimental import pallas as pl


def kernel(X, table, W, b):
    raise NotImplementedError("write your pallas kernel here")


# speedup vs baseline: 8.0799x; 8.0799x over previous
import functools

import jax
import jax.numpy as jnp
from jax import lax
from jax.experimental import pallas as pl
from jax.experimental.pallas import tpu as pltpu
from jax.experimental.pallas import tpu_sc as plsc

_NC = 2
_NS = 16
_NW = _NC * _NS

_EMBED = 128
_CHUNK = 128


def _proj_body(table_ref, w_ref, b_ref, out_ref):
    out_ref[...] = (
        jnp.dot(table_ref[...], w_ref[...], preferred_element_type=jnp.float32)
        + b_ref[...]
    )


def _project_table(table, W, b):
    V, _ = table.shape
    E = W.shape[1]
    return pl.pallas_call(
        _proj_body,
        out_shape=jax.ShapeDtypeStruct((V, E), jnp.float32),
    )(table, W, b.reshape(1, E))


def _make_gather(B, E):
    assert B % (_NW * _CHUNK) == 0
    b_per_w = B // _NW
    n_chunks = b_per_w // _CHUNK
    mesh = plsc.VectorSubcoreMesh(
        core_axis_name="c", subcore_axis_name="s",
        num_cores=_NC, num_subcores=_NS,
    )

    @functools.partial(
        pl.kernel,
        mesh=mesh,
        out_type=jax.ShapeDtypeStruct((B, E), jnp.float32),
        scratch_types=[
            pltpu.VMEM((_CHUNK,), jnp.int32),
            pltpu.VMEM((_CHUNK, E), jnp.float32),
            pltpu.SemaphoreType.DMA,
        ],
    )
    def gather_kernel(idx_hbm, proj_hbm, out_hbm, idx_v, rows_v, sem):
        wid = lax.axis_index("s") * _NC + lax.axis_index("c")
        base = wid * b_per_w

        def step(i, carry):
            off = base + i * _CHUNK
            pltpu.sync_copy(idx_hbm.at[pl.ds(off, _CHUNK)], idx_v)
            pltpu.async_copy(proj_hbm.at[idx_v], rows_v, sem).wait()
            pltpu.sync_copy(rows_v, out_hbm.at[pl.ds(off, _CHUNK)])
            return carry

        lax.fori_loop(0, n_chunks, step, 0)

    return gather_kernel


def kernel(X, table, W, b):
    Bt, S = X.shape
    proj = _project_table(table, W, b)
    idx = X.reshape(-1).astype(jnp.int32)
    out = _make_gather(idx.shape[0], _EMBED)(idx, proj)
    return out.reshape(Bt, S, _EMBED)


# speedup vs baseline: 13.9746x; 1.7295x over previous
import functools

import jax
import jax.numpy as jnp
from jax import lax
from jax.experimental import pallas as pl
from jax.experimental.pallas import tpu as pltpu
from jax.experimental.pallas import tpu_sc as plsc

_NC = 2
_NS = 16
_NW = _NC * _NS

_EMBED = 128
_CHUNK = 128


def _proj_body(table_ref, w_ref, b_ref, out_ref):
    out_ref[...] = (
        jnp.dot(table_ref[...], w_ref[...], preferred_element_type=jnp.float32)
        + b_ref[...]
    )


def _project_table(table, W, b):
    V, _ = table.shape
    E = W.shape[1]
    return pl.pallas_call(
        _proj_body,
        out_shape=jax.ShapeDtypeStruct((V, E), jnp.float32),
    )(table, W, b.reshape(1, E))


_GROUP = 256
_CPG = _GROUP // _CHUNK


def _make_gather(B, E):
    assert B % (_NW * _GROUP) == 0
    b_per_w = B // _NW
    n_groups = b_per_w // _GROUP
    assert n_groups % 2 == 0 and n_groups >= 4
    mesh = plsc.VectorSubcoreMesh(
        core_axis_name="c", subcore_axis_name="s",
        num_cores=_NC, num_subcores=_NS,
    )

    @functools.partial(
        pl.kernel,
        mesh=mesh,
        out_type=jax.ShapeDtypeStruct((B, E), jnp.float32),
        scratch_types=[
            pltpu.VMEM((b_per_w,), jnp.int32),
            pltpu.VMEM((_GROUP, E), jnp.float32),
            pltpu.VMEM((_GROUP, E), jnp.float32),
            pltpu.SemaphoreType.DMA,
            pltpu.SemaphoreType.DMA,
            pltpu.SemaphoreType.DMA,
            pltpu.SemaphoreType.DMA,
        ],
    )
    def gather_kernel(idx_hbm, proj_hbm, out_hbm, idx_all,
                      rows_a, rows_b, sg0, sg1, ss0, ss1):
        rows = [rows_a, rows_b]
        sem_g = [sg0, sg1]
        sem_st = [ss0, ss1]
        wid = lax.axis_index("s") * _NC + lax.axis_index("c")
        base = wid * b_per_w
        pltpu.sync_copy(idx_hbm.at[pl.ds(base, b_per_w)], idx_all)

        def fire_gathers(g, p):
            for c in range(_CPG):
                off = g * _GROUP + c * _CHUNK
                pltpu.async_copy(
                    proj_hbm.at[idx_all.at[pl.ds(off, _CHUNK)]],
                    rows[p].at[pl.ds(c * _CHUNK, _CHUNK)],
                    sem_g[p])

        def wait_gathers(g, p):
            for c in range(_CPG):
                off = g * _GROUP + c * _CHUNK
                pltpu.make_async_copy(
                    proj_hbm.at[idx_all.at[pl.ds(off, _CHUNK)]],
                    rows[p].at[pl.ds(c * _CHUNK, _CHUNK)],
                    sem_g[p]).wait()

        def fire_store(g, p):
            pltpu.async_copy(
                rows[p], out_hbm.at[pl.ds(base + g * _GROUP, _GROUP)],
                sem_st[p])

        def wait_store(g, p):
            pltpu.make_async_copy(
                rows[p], out_hbm.at[pl.ds(base + g * _GROUP, _GROUP)],
                sem_st[p]).wait()

        fire_gathers(0, 0)
        fire_gathers(1, 1)
        wait_gathers(0, 0)
        fire_store(0, 0)

        def pair(t, carry):
            g = 2 * t
            wait_store(g - 2, 0)
            fire_gathers(g, 0)
            wait_gathers(g - 1, 1)
            fire_store(g - 1, 1)
            wait_store(g - 1, 1)
            fire_gathers(g + 1, 1)
            wait_gathers(g, 0)
            fire_store(g, 0)
            return carry

        lax.fori_loop(1, n_groups // 2, pair, 0)

        g_last = n_groups - 1
        wait_gathers(g_last, 1)
        fire_store(g_last, 1)
        wait_store(g_last - 1, 0)
        wait_store(g_last, 1)

    return gather_kernel


def kernel(X, table, W, b):
    Bt, S = X.shape
    proj = _project_table(table, W, b)
    idx = X.reshape(-1).astype(jnp.int32)
    out = _make_gather(idx.shape[0], _EMBED)(idx, proj)
    return out.reshape(Bt, S, _EMBED)
